# baseline (device time: 145626 ns/iter reference)
import numpy as np
import jax
import jax.numpy as jnp
from jax import lax
from jax.experimental import pallas as pl
from jax.experimental.pallas import tpu as pltpu

N_DEV = 4
B = 2
SQ_G = 1024
SQ_L = SQ_G // N_DEV
D_MODEL = 768
H_L = 4
DH = 64
HD_L = H_L * DH
SCALE = 0.125


def _rope_tables():
    inv = 1.0 / (10000.0 ** (np.arange(0, DH, 2) / DH))
    pos = np.arange(SQ_G)[:, None] * inv[None, :]
    cos = np.repeat(np.cos(pos), 2, axis=-1)
    sin = np.repeat(np.sin(pos), 2, axis=-1)
    cos_t = np.tile(cos, (1, H_L)).astype(np.float32)
    sin_t = np.tile(sin, (1, H_L)).astype(np.float32)
    p64 = np.zeros((DH, DH), dtype=np.float32)
    for k in range(DH // 2):
        p64[2 * k + 1, 2 * k] = -1.0
        p64[2 * k, 2 * k + 1] = 1.0
    perm = np.kron(np.eye(H_L, dtype=np.float32), p64)
    return jnp.asarray(cos_t), jnp.asarray(sin_t), jnp.asarray(perm)


def kernel(x, Wq, Wk, Wv, Wo):
    cos_t, sin_t, perm = _rope_tables()

    def body(x_ref, wq_ref, wk_ref, wv_ref, wo_ref, cos_ref, sin_ref, p_ref,
             out_ref,
             xg_ref, q_ref, k_ref, v_ref, ctx_ref, sbuf_ref, rbuf_ref,
             ag_send, ag_recv, rs_send, rs_recv):
        me = lax.axis_index("i")
        left = (me - 1) % N_DEV
        right = (me + 1) % N_DEV

        barrier = pltpu.get_barrier_semaphore()
        for nbr in (left, right):
            pl.semaphore_signal(
                barrier, inc=1,
                device_id=(nbr,), device_id_type=pl.DeviceIdType.MESH,
            )
        pl.semaphore_wait(barrier, 2)

        xg_ref[:, pl.ds(me * SQ_L, SQ_L), :] = x_ref[...]
        for h in range(N_DEV - 1):
            org = (me - h) % N_DEV
            rdma = pltpu.make_async_remote_copy(
                src_ref=xg_ref.at[:, pl.ds(org * SQ_L, SQ_L), :],
                dst_ref=xg_ref.at[:, pl.ds(org * SQ_L, SQ_L), :],
                send_sem=ag_send.at[h],
                recv_sem=ag_recv.at[h],
                device_id=(right,),
                device_id_type=pl.DeviceIdType.MESH,
            )
            rdma.start()
            rdma.wait()

        for b in range(B):
            xb = xg_ref[b]
            cos = cos_ref[...]
            sin = sin_ref[...]
            p = p_ref[...]
            q = jnp.dot(xb, wq_ref[...], preferred_element_type=jnp.float32)
            k = jnp.dot(xb, wk_ref[...], preferred_element_type=jnp.float32)
            q_ref[b] = q * cos + jnp.dot(q, p, preferred_element_type=jnp.float32) * sin
            k_ref[b] = k * cos + jnp.dot(k, p, preferred_element_type=jnp.float32) * sin
            v_ref[b] = jnp.dot(xb, wv_ref[...], preferred_element_type=jnp.float32)

        for b in range(B):
            for h in range(H_L):
                sl = slice(h * DH, (h + 1) * DH)
                qh = q_ref[b, :, sl]
                kh = k_ref[b, :, sl]
                vh = v_ref[b, :, sl]
                s = lax.dot_general(
                    qh, kh, (((1,), (1,)), ((), ())),
                    preferred_element_type=jnp.float32,
                ) * SCALE
                m = jnp.max(s, axis=-1, keepdims=True)
                w = jnp.exp(s - m)
                w = w / jnp.sum(w, axis=-1, keepdims=True)
                ctx_ref[b, :, sl] = jnp.dot(
                    w, vh, preferred_element_type=jnp.float32)

        for s_ in range(N_DEV - 1):
            c = (me + s_ + 1) % N_DEV
            for b in range(B):
                acc = jnp.dot(
                    ctx_ref[b, pl.ds(c * SQ_L, SQ_L), :], wo_ref[...],
                    preferred_element_type=jnp.float32)
                if s_ > 0:
                    acc = acc + rbuf_ref[s_ - 1, b]
                sbuf_ref[s_, b] = acc
            rdma = pltpu.make_async_remote_copy(
                src_ref=sbuf_ref.at[s_],
                dst_ref=rbuf_ref.at[s_],
                send_sem=rs_send.at[s_],
                recv_sem=rs_recv.at[s_],
                device_id=(left,),
                device_id_type=pl.DeviceIdType.MESH,
            )
            rdma.start()
            rdma.wait()

        for b in range(B):
            own = jnp.dot(
                ctx_ref[b, pl.ds(me * SQ_L, SQ_L), :], wo_ref[...],
                preferred_element_type=jnp.float32)
            out_ref[b] = own + rbuf_ref[N_DEV - 2, b]

    return pl.pallas_call(
        body,
        out_shape=jax.ShapeDtypeStruct((B, SQ_L, D_MODEL), jnp.float32),
        in_specs=[pl.BlockSpec(memory_space=pltpu.VMEM)] * 8,
        out_specs=pl.BlockSpec(memory_space=pltpu.VMEM),
        scratch_shapes=[
            pltpu.VMEM((B, SQ_G, D_MODEL), jnp.float32),
            pltpu.VMEM((B, SQ_G, HD_L), jnp.float32),
            pltpu.VMEM((B, SQ_G, HD_L), jnp.float32),
            pltpu.VMEM((B, SQ_G, HD_L), jnp.float32),
            pltpu.VMEM((B, SQ_G, HD_L), jnp.float32),
            pltpu.VMEM((N_DEV - 1, B, SQ_L, D_MODEL), jnp.float32),
            pltpu.VMEM((N_DEV - 1, B, SQ_L, D_MODEL), jnp.float32),
            pltpu.SemaphoreType.DMA((N_DEV - 1,)),
            pltpu.SemaphoreType.DMA((N_DEV - 1,)),
            pltpu.SemaphoreType.DMA((N_DEV - 1,)),
            pltpu.SemaphoreType.DMA((N_DEV - 1,)),
        ],
        compiler_params=pltpu.CompilerParams(collective_id=0),
    )(x, Wq, Wk, Wv, Wo, cos_t, sin_t, perm)


# device time: 93100 ns/iter; 1.5642x vs baseline; 1.5642x over previous
import numpy as np
import jax
import jax.numpy as jnp
from jax import lax
from jax.experimental import pallas as pl
from jax.experimental.pallas import tpu as pltpu

N_DEV = 4
B = 2
SQ_G = 1024
SQ_L = SQ_G // N_DEV
D_MODEL = 768
H_L = 4
DH = 64
HD_L = H_L * DH
SCALE = 0.125


def _rope_tables():
    inv = 1.0 / (10000.0 ** (np.arange(0, DH, 2) / DH))
    pos = np.arange(SQ_G)[:, None] * inv[None, :]
    cos = np.repeat(np.cos(pos), 2, axis=-1)
    sin = np.repeat(np.sin(pos), 2, axis=-1)
    cos_t = np.tile(cos, (1, H_L)).astype(np.float32)
    sin_t = np.tile(sin, (1, H_L)).astype(np.float32)
    p64 = np.zeros((DH, DH), dtype=np.float32)
    for k in range(DH // 2):
        p64[2 * k + 1, 2 * k] = -1.0
        p64[2 * k, 2 * k + 1] = 1.0
    perm = np.kron(np.eye(H_L, dtype=np.float32), p64)
    return jnp.asarray(cos_t), jnp.asarray(sin_t), jnp.asarray(perm)


def kernel(x, Wq, Wk, Wv, Wo):
    cos_t, sin_t, perm = _rope_tables()

    def body(x_ref, wq_ref, wk_ref, wv_ref, wo_ref, cos_ref, sin_ref, p_ref,
             out_ref,
             xg_ref, q_ref, k_ref, v_ref,
             sbl_ref, rbl_ref, sbr_ref, rbr_ref,
             agr_send, agr_recv, agl_send, agl_recv,
             rsl_send, rsl_recv, rsr_send, rsr_recv):
        me = lax.axis_index("i")
        left = (me - 1) % N_DEV
        right = (me + 1) % N_DEV

        barrier = pltpu.get_barrier_semaphore()
        for nbr in (left, right):
            pl.semaphore_signal(
                barrier, inc=1,
                device_id=(nbr,), device_id_type=pl.DeviceIdType.MESH,
            )
        pl.semaphore_wait(barrier, 2)

        def chunk_rdma(src, dst, ssem, rsem, to):
            return pltpu.make_async_remote_copy(
                src_ref=src, dst_ref=dst, send_sem=ssem, recv_sem=rsem,
                device_id=(to,), device_id_type=pl.DeviceIdType.MESH,
            )

        def xg_at(c):
            return xg_ref.at[:, pl.ds(c * SQ_L, SQ_L), :]

        def qkv_chunk(c):
            rows = pl.ds(c * SQ_L, SQ_L)
            cos = cos_ref[rows, :]
            sin = sin_ref[rows, :]
            p = p_ref[...]
            for b in range(B):
                xb = xg_ref[b, rows, :]
                q = jnp.dot(xb, wq_ref[...],
                            preferred_element_type=jnp.float32)
                k = jnp.dot(xb, wk_ref[...],
                            preferred_element_type=jnp.float32)
                q_ref[b, rows, :] = q * cos + jnp.dot(
                    q, p, preferred_element_type=jnp.float32) * sin
                k_ref[b, rows, :] = k * cos + jnp.dot(
                    k, p, preferred_element_type=jnp.float32) * sin
                v_ref[b, rows, :] = jnp.dot(
                    xb, wv_ref[...], preferred_element_type=jnp.float32)

        xg_ref[:, pl.ds(me * SQ_L, SQ_L), :] = x_ref[...]
        r0 = chunk_rdma(xg_at(me), xg_at(me), agr_send.at[0],
                        agr_recv.at[0], right)
        r0.start()
        l0 = chunk_rdma(xg_at(me), xg_at(me), agl_send.at[0],
                        agl_recv.at[0], left)
        l0.start()
        qkv_chunk(me)

        r0.wait_recv()
        r1 = chunk_rdma(xg_at(left), xg_at(left), agr_send.at[1],
                        agr_recv.at[1], right)
        r1.start()
        qkv_chunk(left)

        l0.wait_recv()
        qkv_chunk(right)

        r1.wait_recv()
        qkv_chunk((me + 2) % N_DEV)

        def partial_chunk(c):
            rows = pl.ds(c * SQ_L, SQ_L)
            outs = []
            for b in range(B):
                ctxs = []
                for h in range(H_L):
                    hsl = slice(h * DH, (h + 1) * DH)
                    qh = q_ref[b, rows, hsl]
                    kh = k_ref[b, :, hsl]
                    vh = v_ref[b, :, hsl]
                    s = lax.dot_general(
                        qh, kh, (((1,), (1,)), ((), ())),
                        preferred_element_type=jnp.float32,
                    ) * SCALE
                    m = jnp.max(s, axis=-1, keepdims=True)
                    w = jnp.exp(s - m)
                    w = w / jnp.sum(w, axis=-1, keepdims=True)
                    ctxs.append(jnp.dot(w, vh,
                                        preferred_element_type=jnp.float32))
                ctx = jnp.concatenate(ctxs, axis=-1)
                outs.append(jnp.dot(ctx, wo_ref[...],
                                    preferred_element_type=jnp.float32))
            return outs

        for b, pb in enumerate(partial_chunk((me + 2) % N_DEV)):
            sbl_ref[0, b] = pb
        ls0 = chunk_rdma(sbl_ref.at[0], rbl_ref.at[0], rsl_send.at[0],
                         rsl_recv.at[0], left)
        ls0.start()

        for b, pb in enumerate(partial_chunk(right)):
            sbr_ref[0, b] = pb
        rs0 = chunk_rdma(sbr_ref.at[0], rbr_ref.at[0], rsr_send.at[0],
                         rsr_recv.at[0], right)
        rs0.start()

        p_m1 = partial_chunk(left)
        ls0.wait_recv()
        for b in range(B):
            sbl_ref[1, b] = p_m1[b] + rbl_ref[0, b]
        ls1 = chunk_rdma(sbl_ref.at[1], rbl_ref.at[1], rsl_send.at[1],
                         rsl_recv.at[1], left)
        ls1.start()

        p_own = partial_chunk(me)
        ls1.wait_recv()
        rs0.wait_recv()
        for b in range(B):
            out_ref[b] = p_own[b] + rbl_ref[1, b] + rbr_ref[0, b]

        for d in (r0, r1, l0, ls0, ls1, rs0):
            d.wait_send()

    return pl.pallas_call(
        body,
        out_shape=jax.ShapeDtypeStruct((B, SQ_L, D_MODEL), jnp.float32),
        in_specs=[pl.BlockSpec(memory_space=pltpu.VMEM)] * 8,
        out_specs=pl.BlockSpec(memory_space=pltpu.VMEM),
        scratch_shapes=[
            pltpu.VMEM((B, SQ_G, D_MODEL), jnp.float32),
            pltpu.VMEM((B, SQ_G, HD_L), jnp.float32),
            pltpu.VMEM((B, SQ_G, HD_L), jnp.float32),
            pltpu.VMEM((B, SQ_G, HD_L), jnp.float32),
            pltpu.VMEM((2, B, SQ_L, D_MODEL), jnp.float32),
            pltpu.VMEM((2, B, SQ_L, D_MODEL), jnp.float32),
            pltpu.VMEM((1, B, SQ_L, D_MODEL), jnp.float32),
            pltpu.VMEM((1, B, SQ_L, D_MODEL), jnp.float32),
            pltpu.SemaphoreType.DMA((2,)),
            pltpu.SemaphoreType.DMA((2,)),
            pltpu.SemaphoreType.DMA((1,)),
            pltpu.SemaphoreType.DMA((1,)),
            pltpu.SemaphoreType.DMA((2,)),
            pltpu.SemaphoreType.DMA((2,)),
            pltpu.SemaphoreType.DMA((1,)),
            pltpu.SemaphoreType.DMA((1,)),
        ],
        compiler_params=pltpu.CompilerParams(collective_id=0),
    )(x, Wq, Wk, Wv, Wo, cos_t, sin_t, perm)


# device time: 64636 ns/iter; 2.2530x vs baseline; 1.4404x over previous
import numpy as np
import jax
import jax.numpy as jnp
from jax import lax
from jax.experimental import pallas as pl
from jax.experimental.pallas import tpu as pltpu

N_DEV = 4
B = 2
SQ_G = 1024
SQ_L = SQ_G // N_DEV
D_MODEL = 768
H_L = 4
DH = 64
HD_L = H_L * DH
SCALE = 0.125
BF = jnp.bfloat16
F32 = jnp.float32


def _rope_tables():
    inv = 1.0 / (10000.0 ** (np.arange(0, DH, 2) / DH))
    pos = np.arange(SQ_G)[:, None] * inv[None, :]
    cos = np.repeat(np.cos(pos), 2, axis=-1)
    sin = np.repeat(np.sin(pos), 2, axis=-1)
    cos_t = np.tile(cos, (1, H_L)).astype(np.float32)
    sin_t = np.tile(sin, (1, H_L)).astype(np.float32)
    p64 = np.zeros((DH, DH), dtype=np.float32)
    for k in range(DH // 2):
        p64[2 * k + 1, 2 * k] = -1.0
        p64[2 * k, 2 * k + 1] = 1.0
    perm = np.kron(np.eye(H_L, dtype=np.float32), p64)
    return (jnp.asarray(cos_t), jnp.asarray(sin_t),
            jnp.asarray(perm, dtype=BF))


def kernel(x, Wq, Wk, Wv, Wo):
    cos_t, sin_t, perm = _rope_tables()

    def body(x_ref, wq_ref, wk_ref, wv_ref, wo_ref, cos_ref, sin_ref, p_ref,
             out_ref,
             xg_ref, q_ref, k_ref, v_ref,
             sbl_ref, rbl_ref, sbr_ref, rbr_ref,
             agr_send, agr_recv, agl_send, agl_recv,
             rsl_send, rsl_recv, rsr_send, rsr_recv):
        me = lax.axis_index("i")
        left = (me - 1) % N_DEV
        right = (me + 1) % N_DEV

        barrier = pltpu.get_barrier_semaphore()
        for nbr in (left, right):
            pl.semaphore_signal(
                barrier, inc=1,
                device_id=(nbr,), device_id_type=pl.DeviceIdType.MESH,
            )
        pl.semaphore_wait(barrier, 2)

        def chunk_rdma(src, dst, ssem, rsem, to):
            return pltpu.make_async_remote_copy(
                src_ref=src, dst_ref=dst, send_sem=ssem, recv_sem=rsem,
                device_id=(to,), device_id_type=pl.DeviceIdType.MESH,
            )

        def xg_at(c):
            return xg_ref.at[:, pl.ds(c * SQ_L, SQ_L), :]

        wq_bf = wq_ref[...].astype(BF)
        wk_bf = wk_ref[...].astype(BF)
        wv_bf = wv_ref[...].astype(BF)
        wo_bf = wo_ref[...].astype(BF)

        def qkv_chunk(c):
            rows = pl.ds(c * SQ_L, SQ_L)
            cos = cos_ref[rows, :]
            sin = sin_ref[rows, :]
            p = p_ref[...]
            for b in range(B):
                xb = xg_ref[b, rows, :]
                q = jnp.dot(xb, wq_bf, preferred_element_type=F32)
                k = jnp.dot(xb, wk_bf, preferred_element_type=F32)
                qr = jnp.dot(q.astype(BF), p, preferred_element_type=F32)
                kr = jnp.dot(k.astype(BF), p, preferred_element_type=F32)
                q_ref[b, rows, :] = (q * cos + qr * sin).astype(BF)
                k_ref[b, rows, :] = (k * cos + kr * sin).astype(BF)
                v_ref[b, rows, :] = jnp.dot(
                    xb, wv_bf, preferred_element_type=F32).astype(BF)

        xg_ref[:, pl.ds(me * SQ_L, SQ_L), :] = x_ref[...].astype(BF)
        r0 = chunk_rdma(xg_at(me), xg_at(me), agr_send.at[0],
                        agr_recv.at[0], right)
        r0.start()
        l0 = chunk_rdma(xg_at(me), xg_at(me), agl_send.at[0],
                        agl_recv.at[0], left)
        l0.start()
        qkv_chunk(me)

        r0.wait_recv()
        r1 = chunk_rdma(xg_at(left), xg_at(left), agr_send.at[1],
                        agr_recv.at[1], right)
        r1.start()
        qkv_chunk(left)

        l0.wait_recv()
        qkv_chunk(right)

        r1.wait_recv()
        qkv_chunk((me + 2) % N_DEV)

        def partial_chunk(c):
            rows = pl.ds(c * SQ_L, SQ_L)
            outs = []
            for b in range(B):
                ctxs = []
                for h in range(H_L):
                    hsl = slice(h * DH, (h + 1) * DH)
                    qh = q_ref[b, rows, hsl]
                    kh = k_ref[b, :, hsl]
                    vh = v_ref[b, :, hsl]
                    s = lax.dot_general(
                        qh, kh, (((1,), (1,)), ((), ())),
                        preferred_element_type=F32,
                    ) * SCALE
                    m = jnp.max(s, axis=-1, keepdims=True)
                    w = jnp.exp(s - m)
                    w = (w / jnp.sum(w, axis=-1, keepdims=True)).astype(BF)
                    ctxs.append(jnp.dot(w, vh, preferred_element_type=F32))
                ctx = jnp.concatenate(ctxs, axis=-1).astype(BF)
                outs.append(jnp.dot(ctx, wo_bf, preferred_element_type=F32))
            return outs

        for b, pb in enumerate(partial_chunk((me + 2) % N_DEV)):
            sbl_ref[0, b] = pb.astype(BF)
        ls0 = chunk_rdma(sbl_ref.at[0], rbl_ref.at[0], rsl_send.at[0],
                         rsl_recv.at[0], left)
        ls0.start()

        for b, pb in enumerate(partial_chunk(right)):
            sbr_ref[0, b] = pb.astype(BF)
        rs0 = chunk_rdma(sbr_ref.at[0], rbr_ref.at[0], rsr_send.at[0],
                         rsr_recv.at[0], right)
        rs0.start()

        p_m1 = partial_chunk(left)
        ls0.wait_recv()
        for b in range(B):
            sbl_ref[1, b] = (p_m1[b] + rbl_ref[0, b]).astype(BF)
        ls1 = chunk_rdma(sbl_ref.at[1], rbl_ref.at[1], rsl_send.at[1],
                         rsl_recv.at[1], left)
        ls1.start()

        p_own = partial_chunk(me)
        ls1.wait_recv()
        rs0.wait_recv()
        for b in range(B):
            out_ref[b] = (p_own[b] + rbl_ref[1, b].astype(F32)
                          + rbr_ref[0, b].astype(F32))

        for d in (r0, r1, l0, ls0, ls1, rs0):
            d.wait_send()

    return pl.pallas_call(
        body,
        out_shape=jax.ShapeDtypeStruct((B, SQ_L, D_MODEL), jnp.float32),
        in_specs=[pl.BlockSpec(memory_space=pltpu.VMEM)] * 8,
        out_specs=pl.BlockSpec(memory_space=pltpu.VMEM),
        scratch_shapes=[
            pltpu.VMEM((B, SQ_G, D_MODEL), BF),
            pltpu.VMEM((B, SQ_G, HD_L), BF),
            pltpu.VMEM((B, SQ_G, HD_L), BF),
            pltpu.VMEM((B, SQ_G, HD_L), BF),
            pltpu.VMEM((2, B, SQ_L, D_MODEL), BF),
            pltpu.VMEM((2, B, SQ_L, D_MODEL), BF),
            pltpu.VMEM((1, B, SQ_L, D_MODEL), BF),
            pltpu.VMEM((1, B, SQ_L, D_MODEL), BF),
            pltpu.SemaphoreType.DMA((2,)),
            pltpu.SemaphoreType.DMA((2,)),
            pltpu.SemaphoreType.DMA((1,)),
            pltpu.SemaphoreType.DMA((1,)),
            pltpu.SemaphoreType.DMA((2,)),
            pltpu.SemaphoreType.DMA((2,)),
            pltpu.SemaphoreType.DMA((1,)),
            pltpu.SemaphoreType.DMA((1,)),
        ],
        compiler_params=pltpu.CompilerParams(collective_id=0),
    )(x, Wq, Wk, Wv, Wo, cos_t, sin_t, perm)


# device time: 57303 ns/iter; 2.5413x vs baseline; 1.1280x over previous
import numpy as np
import jax
import jax.numpy as jnp
from jax import lax
from jax.experimental import pallas as pl
from jax.experimental.pallas import tpu as pltpu

N_DEV = 4
B = 2
SQ_G = 1024
SQ_L = SQ_G // N_DEV
D_MODEL = 768
H_L = 4
DH = 64
HD_L = H_L * DH
SCALE = 0.125
BF = jnp.bfloat16
F32 = jnp.float32


def _rope_tables():
    inv = 1.0 / (10000.0 ** (np.arange(0, DH, 2) / DH))
    pos = np.arange(SQ_G)[:, None] * inv[None, :]
    cos = np.repeat(np.cos(pos), 2, axis=-1)
    sin = np.repeat(np.sin(pos), 2, axis=-1)
    cos_t = np.tile(cos, (1, H_L)).astype(np.float32)
    sin_t = np.tile(sin, (1, H_L)).astype(np.float32)
    p64 = np.zeros((DH, DH), dtype=np.float32)
    for k in range(DH // 2):
        p64[2 * k + 1, 2 * k] = -1.0
        p64[2 * k, 2 * k + 1] = 1.0
    perm = np.kron(np.eye(H_L, dtype=np.float32), p64)
    return (jnp.asarray(cos_t), jnp.asarray(sin_t),
            jnp.asarray(perm, dtype=BF))


def kernel(x, Wq, Wk, Wv, Wo):
    cos_t, sin_t, perm = _rope_tables()

    def body(x_ref, wq_ref, wk_ref, wv_ref, wo_ref, cos_ref, sin_ref, p_ref,
             out_ref,
             xg_ref, q_ref, k_ref, v_ref,
             sbl_ref, rbl_ref, sbr_ref, rbr_ref,
             agr_send, agr_recv, agl_send, agl_recv,
             rsl_send, rsl_recv, rsr_send, rsr_recv):
        me = lax.axis_index("i")
        left = (me - 1) % N_DEV
        right = (me + 1) % N_DEV

        barrier = pltpu.get_barrier_semaphore()
        for nbr in (left, right):
            pl.semaphore_signal(
                barrier, inc=1,
                device_id=(nbr,), device_id_type=pl.DeviceIdType.MESH,
            )
        pl.semaphore_wait(barrier, 2)

        def chunk_rdma(src, dst, ssem, rsem, to):
            return pltpu.make_async_remote_copy(
                src_ref=src, dst_ref=dst, send_sem=ssem, recv_sem=rsem,
                device_id=(to,), device_id_type=pl.DeviceIdType.MESH,
            )

        def xg_at(c):
            return xg_ref.at[:, pl.ds(c * SQ_L, SQ_L), :]

        wq_bf = wq_ref[...].astype(BF)
        wk_bf = wk_ref[...].astype(BF)
        wv_bf = wv_ref[...].astype(BF)
        wo_bf = wo_ref[...].astype(BF)

        def qkv_chunk(c):
            rows = pl.ds(c * SQ_L, SQ_L)
            cos = cos_ref[rows, :]
            sin = sin_ref[rows, :]
            p = p_ref[...]
            for b in range(B):
                xb = xg_ref[b, rows, :]
                q = jnp.dot(xb, wq_bf, preferred_element_type=F32)
                k = jnp.dot(xb, wk_bf, preferred_element_type=F32)
                qr = jnp.dot(q.astype(BF), p, preferred_element_type=F32)
                kr = jnp.dot(k.astype(BF), p, preferred_element_type=F32)
                q_ref[b, rows, :] = ((q * cos + qr * sin) * SCALE).astype(BF)
                k_ref[b, rows, :] = (k * cos + kr * sin).astype(BF)
                v_ref[b, rows, :] = jnp.dot(
                    xb, wv_bf, preferred_element_type=F32).astype(BF)

        xg_ref[:, pl.ds(me * SQ_L, SQ_L), :] = x_ref[...].astype(BF)
        r0 = chunk_rdma(xg_at(me), xg_at(me), agr_send.at[0],
                        agr_recv.at[0], right)
        r0.start()
        l0 = chunk_rdma(xg_at(me), xg_at(me), agl_send.at[0],
                        agl_recv.at[0], left)
        l0.start()
        qkv_chunk(me)

        r0.wait_recv()
        r1 = chunk_rdma(xg_at(left), xg_at(left), agr_send.at[1],
                        agr_recv.at[1], right)
        r1.start()
        qkv_chunk(left)

        l0.wait_recv()
        qkv_chunk(right)

        r1.wait_recv()
        qkv_chunk((me + 2) % N_DEV)

        def partial_chunk(c):
            rows = pl.ds(c * SQ_L, SQ_L)
            outs = []
            for b in range(B):
                ctxs = []
                for h in range(H_L):
                    hsl = slice(h * DH, (h + 1) * DH)
                    qh = q_ref[b, rows, hsl]
                    kh = k_ref[b, :, hsl]
                    vh = v_ref[b, :, hsl]
                    s = lax.dot_general(
                        qh, kh, (((1,), (1,)), ((), ())),
                        preferred_element_type=F32,
                    )
                    e = jnp.exp(s)
                    r = 1.0 / jnp.sum(e, axis=-1, keepdims=True)
                    ctxs.append(jnp.dot(e.astype(BF), vh,
                                        preferred_element_type=F32) * r)
                ctx = jnp.concatenate(ctxs, axis=-1).astype(BF)
                outs.append(jnp.dot(ctx, wo_bf, preferred_element_type=F32))
            return outs

        for b, pb in enumerate(partial_chunk((me + 2) % N_DEV)):
            sbl_ref[0, b] = pb.astype(BF)
        ls0 = chunk_rdma(sbl_ref.at[0], rbl_ref.at[0], rsl_send.at[0],
                         rsl_recv.at[0], left)
        ls0.start()

        for b, pb in enumerate(partial_chunk(right)):
            sbr_ref[0, b] = pb.astype(BF)
        rs0 = chunk_rdma(sbr_ref.at[0], rbr_ref.at[0], rsr_send.at[0],
                         rsr_recv.at[0], right)
        rs0.start()

        p_m1 = partial_chunk(left)
        ls0.wait_recv()
        for b in range(B):
            sbl_ref[1, b] = (p_m1[b] + rbl_ref[0, b]).astype(BF)
        ls1 = chunk_rdma(sbl_ref.at[1], rbl_ref.at[1], rsl_send.at[1],
                         rsl_recv.at[1], left)
        ls1.start()

        p_own = partial_chunk(me)
        ls1.wait_recv()
        rs0.wait_recv()
        for b in range(B):
            out_ref[b] = (p_own[b] + rbl_ref[1, b].astype(F32)
                          + rbr_ref[0, b].astype(F32))

        for d in (r0, r1, l0, ls0, ls1, rs0):
            d.wait_send()

    return pl.pallas_call(
        body,
        out_shape=jax.ShapeDtypeStruct((B, SQ_L, D_MODEL), jnp.float32),
        in_specs=[pl.BlockSpec(memory_space=pltpu.VMEM)] * 8,
        out_specs=pl.BlockSpec(memory_space=pltpu.VMEM),
        scratch_shapes=[
            pltpu.VMEM((B, SQ_G, D_MODEL), BF),
            pltpu.VMEM((B, SQ_G, HD_L), BF),
            pltpu.VMEM((B, SQ_G, HD_L), BF),
            pltpu.VMEM((B, SQ_G, HD_L), BF),
            pltpu.VMEM((2, B, SQ_L, D_MODEL), BF),
            pltpu.VMEM((2, B, SQ_L, D_MODEL), BF),
            pltpu.VMEM((1, B, SQ_L, D_MODEL), BF),
            pltpu.VMEM((1, B, SQ_L, D_MODEL), BF),
            pltpu.SemaphoreType.DMA((2,)),
            pltpu.SemaphoreType.DMA((2,)),
            pltpu.SemaphoreType.DMA((1,)),
            pltpu.SemaphoreType.DMA((1,)),
            pltpu.SemaphoreType.DMA((2,)),
            pltpu.SemaphoreType.DMA((2,)),
            pltpu.SemaphoreType.DMA((1,)),
            pltpu.SemaphoreType.DMA((1,)),
        ],
        compiler_params=pltpu.CompilerParams(collective_id=0),
    )(x, Wq, Wk, Wv, Wo, cos_t, sin_t, perm)
